# baseline (speedup 1.0000x reference)
import jax
import jax.numpy as jnp
from jax import lax
from jax.experimental import pallas as pl

H, S, D = 12, 2048, 64
NUM_LAYERS = 24
ROWS = H * D
ROW_CHUNK = 64


def _copy_body(*refs):
    ins = refs[:NUM_LAYERS]
    outs = refs[NUM_LAYERS:]
    for k in range(NUM_LAYERS):
        outs[k][...] = ins[k][...][None]


def _topk_body(logits_ref, save_id_ref, bz_ref, save_out_ref, prob_ref,
               ids_ref, max_ref):
    row = logits_ref[...]
    v = row.shape[-1]
    m = jnp.max(row)
    s = jnp.sum(jnp.exp(row - m))
    lse = jnp.log(s) + m
    iota = lax.broadcasted_iota(jnp.int32, row.shape, 1)
    neg_inf = jnp.float32(-jnp.inf)
    cur = row
    vals, idxs = [], []
    for _ in range(save_id_ref.shape[0]):
        vj = jnp.max(cur)
        ij = jnp.min(jnp.where(cur >= vj, iota, v))
        vals.append(vj.reshape(1, 1))
        idxs.append(ij.reshape(1, 1))
        cur = jnp.where(iota == ij, neg_inf, cur)
    ids_col = jnp.concatenate(idxs, axis=0) + bz_ref[0, 0]
    prob_col = jnp.concatenate(vals, axis=0) - lse
    ids_ref[...] = ids_col
    prob_ref[...] = prob_col
    save_out_ref[:, 0:1] = save_id_ref[...]
    save_out_ref[:, 1:2] = ids_col
    max_ref[...] = ids_col[0:1, :]


def kernel(kv0, kv1, kv2, kv3, kv4, kv5, kv6, kv7, kv8, kv9, kv10, kv11,
           kv12, kv13, kv14, kv15, kv16, kv17, kv18, kv19, kv20, kv21,
           kv22, kv23, logits, save_id, beam_size):
    kvs = [kv0, kv1, kv2, kv3, kv4, kv5, kv6, kv7, kv8, kv9, kv10, kv11,
           kv12, kv13, kv14, kv15, kv16, kv17, kv18, kv19, kv20, kv21,
           kv22, kv23]
    beam = save_id.shape[0]
    vocab = logits.shape[-1]

    flat_kvs = [kv.reshape(H, S, D).swapaxes(1, 2).reshape(ROWS, S)
                for kv in kvs]
    grid = (ROWS // ROW_CHUNK, beam)
    in_spec = pl.BlockSpec((ROW_CHUNK, S), lambda i, b: (i, 0))
    out_spec = pl.BlockSpec((1, ROW_CHUNK, S), lambda i, b: (b, i, 0))
    tiled_flat = pl.pallas_call(
        _copy_body,
        grid=grid,
        in_specs=[in_spec] * NUM_LAYERS,
        out_specs=[out_spec] * NUM_LAYERS,
        out_shape=[jax.ShapeDtypeStruct((beam, ROWS, S), kv.dtype)
                   for kv in kvs],
    )(*flat_kvs)
    tiled = [t.reshape(beam, H, D, S).swapaxes(2, 3) for t in tiled_flat]

    bz = (jnp.asarray(beam_size, jnp.int32) - jnp.int32(beam)).reshape(1, 1)
    save_out, prob, ids, max_idx = pl.pallas_call(
        _topk_body,
        in_specs=[
            pl.BlockSpec((1, vocab), lambda: (0, 0)),
            pl.BlockSpec((beam, 1), lambda: (0, 0)),
            pl.BlockSpec((1, 1), lambda: (0, 0)),
        ],
        out_specs=[
            pl.BlockSpec((beam, 2), lambda: (0, 0)),
            pl.BlockSpec((beam, 1), lambda: (0, 0)),
            pl.BlockSpec((beam, 1), lambda: (0, 0)),
            pl.BlockSpec((1, 1), lambda: (0, 0)),
        ],
        out_shape=[
            jax.ShapeDtypeStruct((beam, 2), jnp.int32),
            jax.ShapeDtypeStruct((beam, 1), jnp.float32),
            jax.ShapeDtypeStruct((beam, 1), jnp.int32),
            jax.ShapeDtypeStruct((1, 1), jnp.int32),
        ],
    )(logits, save_id, bz)

    return (*tiled, save_out, prob, ids, max_idx)


# speedup vs baseline: 1.0500x; 1.0500x over previous
import jax
import jax.numpy as jnp
from jax import lax
from jax.experimental import pallas as pl
from jax.experimental.pallas import tpu as pltpu

H, S, D = 12, 2048, 64
NUM_LAYERS = 24
ROWS = H * D
ROW_CHUNK = 32


def _fused_body(*refs):
    ins = refs[:NUM_LAYERS]
    logits_ref, save_id_ref, bz_ref = refs[NUM_LAYERS:NUM_LAYERS + 3]
    outs = refs[NUM_LAYERS + 3:2 * NUM_LAYERS + 3]
    save_out_ref, prob_ref, ids_ref, max_ref = refs[2 * NUM_LAYERS + 3:]
    for k in range(NUM_LAYERS):
        outs[k][...] = jnp.broadcast_to(ins[k][...][None], outs[k].shape)

    @pl.when(pl.program_id(0) == 0)
    def _():
        _topk_body(logits_ref, save_id_ref, bz_ref, save_out_ref,
                   prob_ref, ids_ref, max_ref)


def _topk_body(logits_ref, save_id_ref, bz_ref, save_out_ref, prob_ref,
               ids_ref, max_ref):
    row = logits_ref[...]
    v = row.shape[-1]
    m = jnp.max(row)
    s = jnp.sum(jnp.exp(row - m))
    lse = jnp.log(s) + m
    iota = lax.broadcasted_iota(jnp.int32, row.shape, 1)
    neg_inf = jnp.float32(-jnp.inf)
    cur = row
    vals, idxs = [], []
    for _ in range(save_id_ref.shape[0]):
        vj = jnp.max(cur)
        ij = jnp.min(jnp.where(cur >= vj, iota, v))
        vals.append(vj.reshape(1, 1))
        idxs.append(ij.reshape(1, 1))
        cur = jnp.where(iota == ij, neg_inf, cur)
    ids_col = jnp.concatenate(idxs, axis=0) + bz_ref[0, 0]
    prob_col = jnp.concatenate(vals, axis=0) - lse
    ids_ref[...] = ids_col
    prob_ref[...] = prob_col
    save_out_ref[:, 0:1] = save_id_ref[...]
    save_out_ref[:, 1:2] = ids_col
    max_ref[...] = ids_col[0:1, :]


def kernel(kv0, kv1, kv2, kv3, kv4, kv5, kv6, kv7, kv8, kv9, kv10, kv11,
           kv12, kv13, kv14, kv15, kv16, kv17, kv18, kv19, kv20, kv21,
           kv22, kv23, logits, save_id, beam_size):
    kvs = [kv0, kv1, kv2, kv3, kv4, kv5, kv6, kv7, kv8, kv9, kv10, kv11,
           kv12, kv13, kv14, kv15, kv16, kv17, kv18, kv19, kv20, kv21,
           kv22, kv23]
    beam = save_id.shape[0]
    vocab = logits.shape[-1]

    flat_kvs = [kv.reshape(H, S, D).swapaxes(1, 2).reshape(ROWS, S)
                for kv in kvs]
    bz = (jnp.asarray(beam_size, jnp.int32) - jnp.int32(beam)).reshape(1, 1)
    grid = (ROWS // ROW_CHUNK,)
    in_spec = pl.BlockSpec((ROW_CHUNK, S), lambda i: (i, 0))
    out_spec = pl.BlockSpec((beam, ROW_CHUNK, S), lambda i: (0, i, 0))
    outputs = pl.pallas_call(
        _fused_body,
        grid=grid,
        in_specs=[in_spec] * NUM_LAYERS + [
            pl.BlockSpec((1, vocab), lambda i: (0, 0)),
            pl.BlockSpec((beam, 1), lambda i: (0, 0)),
            pl.BlockSpec((1, 1), lambda i: (0, 0)),
        ],
        out_specs=[out_spec] * NUM_LAYERS + [
            pl.BlockSpec((beam, 2), lambda i: (0, 0)),
            pl.BlockSpec((beam, 1), lambda i: (0, 0)),
            pl.BlockSpec((beam, 1), lambda i: (0, 0)),
            pl.BlockSpec((1, 1), lambda i: (0, 0)),
        ],
        out_shape=[jax.ShapeDtypeStruct((beam, ROWS, S), kv.dtype)
                   for kv in kvs] + [
            jax.ShapeDtypeStruct((beam, 2), jnp.int32),
            jax.ShapeDtypeStruct((beam, 1), jnp.float32),
            jax.ShapeDtypeStruct((beam, 1), jnp.int32),
            jax.ShapeDtypeStruct((1, 1), jnp.int32),
        ],
    )(*flat_kvs, logits, save_id, bz)
    tiled = [t.reshape(beam, H, D, S).swapaxes(2, 3)
             for t in outputs[:NUM_LAYERS]]
    save_out, prob, ids, max_idx = outputs[NUM_LAYERS:]

    return (*tiled, save_out, prob, ids, max_idx)
